# Initial kernel scaffold; baseline (speedup 1.0000x reference)
import jax
import jax.numpy as jnp
from jax.experimental import pallas as pl


def kernel(token_ids, embedding_mat):
    raise NotImplementedError("write your pallas kernel here")


# speedup vs baseline: 1.1037x; 1.1037x over previous
import functools

import jax
import jax.numpy as jnp
from jax import lax
from jax.experimental import pallas as pl
from jax.experimental.pallas import tpu as pltpu
from jax.experimental.pallas import tpu_sc as plsc

BATCH = 16384
HIST_LEN = 50
EMB_DIM = 32
N_TOKENS = BATCH * HIST_LEN

NUM_WORKERS = 32
PER_WORKER = N_TOKENS // NUM_WORKERS
CHUNK = 1600
NUM_CHUNKS = PER_WORKER // CHUNK


def _gather_body(tok_hbm, tab_hbm, out_hbm, idx_v, rows_v, sem):
    wid = lax.axis_index("s") * 2 + lax.axis_index("c")
    base = wid * PER_WORKER

    def step(g, carry):
        off = base + g * CHUNK
        pltpu.sync_copy(tok_hbm.at[pl.ds(off, CHUNK)], idx_v)
        pltpu.async_copy(tab_hbm.at[idx_v], rows_v, sem).wait()
        pltpu.sync_copy(rows_v, out_hbm.at[pl.ds(off, CHUNK)])
        return carry

    lax.fori_loop(0, NUM_CHUNKS, step, 0)


@jax.jit
def _embedding_lookup(tok_flat, embedding_mat):
    mesh = plsc.VectorSubcoreMesh(core_axis_name="c", subcore_axis_name="s")
    run = pl.kernel(
        _gather_body,
        mesh=mesh,
        out_type=jax.ShapeDtypeStruct((N_TOKENS, EMB_DIM), jnp.float32),
        scratch_types=[
            pltpu.VMEM((CHUNK,), jnp.int32),
            pltpu.VMEM((CHUNK, EMB_DIM), jnp.float32),
            pltpu.SemaphoreType.DMA,
        ],
        compiler_params=pltpu.CompilerParams(use_tc_tiling_on_sc=False),
    )
    return run(tok_flat, embedding_mat)


def kernel(token_ids, embedding_mat):
    tok_flat = token_ids.reshape(-1).astype(jnp.int32)
    out = _embedding_lookup(tok_flat, embedding_mat)
    return out.reshape(token_ids.shape + (EMB_DIM,))


# speedup vs baseline: 1.1123x; 1.0078x over previous
import functools

import jax
import jax.numpy as jnp
from jax import lax
from jax.experimental import pallas as pl
from jax.experimental.pallas import tpu as pltpu
from jax.experimental.pallas import tpu_sc as plsc

BATCH = 16384
HIST_LEN = 50
EMB_DIM = 32
N_TOKENS = BATCH * HIST_LEN

NUM_WORKERS = 32
PER_WORKER = N_TOKENS // NUM_WORKERS
CHUNK = 1600
NUM_CHUNKS = PER_WORKER // CHUNK


def _gather_body(tok_hbm, tab_hbm, out_hbm,
                 idx0, idx1, rows0, rows1,
                 s_i0, s_i1, s_g0, s_g1, s_o0, s_o1):
    wid = lax.axis_index("s") * 2 + lax.axis_index("c")
    base = wid * PER_WORKER
    idxb, rowsb = [idx0, idx1], [rows0, rows1]
    s_i, s_g, s_o = [s_i0, s_i1], [s_g0, s_g1], [s_o0, s_o1]

    def start_idx(g):
        off = base + g * CHUNK
        return pltpu.async_copy(tok_hbm.at[pl.ds(off, CHUNK)], idxb[g & 1],
                                s_i[g & 1])

    def start_gather(g):
        return pltpu.async_copy(tab_hbm.at[idxb[g & 1]], rowsb[g & 1],
                                s_g[g & 1])

    def start_out(g):
        off = base + g * CHUNK
        return pltpu.async_copy(rowsb[g & 1], out_hbm.at[pl.ds(off, CHUNK)],
                                s_o[g & 1])

    ic = [start_idx(0), start_idx(1)]
    ic[0].wait()
    gc = [None, None]
    gc[0] = start_gather(0)
    oc = [None, None]
    for g in range(NUM_CHUNKS):
        b = g & 1
        nb = 1 - b
        if g + 1 < NUM_CHUNKS:
            ic[nb].wait()
            if oc[nb] is not None:
                oc[nb].wait()
            gc[nb] = start_gather(g + 1)
        gc[b].wait()
        if g + 2 < NUM_CHUNKS:
            ic[b] = start_idx(g + 2)
        oc[b] = start_out(g)
    oc[(NUM_CHUNKS - 2) & 1].wait()
    oc[(NUM_CHUNKS - 1) & 1].wait()


@jax.jit
def _embedding_lookup(tok_flat, embedding_mat):
    mesh = plsc.VectorSubcoreMesh(core_axis_name="c", subcore_axis_name="s")
    run = pl.kernel(
        _gather_body,
        mesh=mesh,
        out_type=jax.ShapeDtypeStruct((N_TOKENS, EMB_DIM), jnp.float32),
        scratch_types=[
            pltpu.VMEM((CHUNK,), jnp.int32),
            pltpu.VMEM((CHUNK,), jnp.int32),
            pltpu.VMEM((CHUNK, EMB_DIM), jnp.float32),
            pltpu.VMEM((CHUNK, EMB_DIM), jnp.float32),
            pltpu.SemaphoreType.DMA,
            pltpu.SemaphoreType.DMA,
            pltpu.SemaphoreType.DMA,
            pltpu.SemaphoreType.DMA,
            pltpu.SemaphoreType.DMA,
            pltpu.SemaphoreType.DMA,
        ],
        compiler_params=pltpu.CompilerParams(use_tc_tiling_on_sc=False),
    )
    return run(tok_flat, embedding_mat)


def kernel(token_ids, embedding_mat):
    tok_flat = token_ids.reshape(-1).astype(jnp.int32)
    out = _embedding_lookup(tok_flat, embedding_mat)
    return out.reshape(token_ids.shape + (EMB_DIM,))


# speedup vs baseline: 1.1128x; 1.0004x over previous
import functools

import jax
import jax.numpy as jnp
from jax import lax
from jax.experimental import pallas as pl
from jax.experimental.pallas import tpu as pltpu
from jax.experimental.pallas import tpu_sc as plsc

BATCH = 16384
HIST_LEN = 50
EMB_DIM = 32
N_TOKENS = BATCH * HIST_LEN

NUM_WORKERS = 32
PER_WORKER = N_TOKENS // NUM_WORKERS
CHUNK = 1600
NUM_CHUNKS = PER_WORKER // CHUNK
KSUB = 4
SUB = CHUNK // KSUB


def _gather_body(tok_hbm, tab_hbm, out_hbm,
                 idx0, idx1, rows0, rows1,
                 s_i0, s_i1, s_g0, s_g1, s_o0, s_o1):
    wid = lax.axis_index("s") * 2 + lax.axis_index("c")
    base = wid * PER_WORKER
    idxb, rowsb = [idx0, idx1], [rows0, rows1]
    s_i, s_g, s_o = [s_i0, s_i1], [s_g0, s_g1], [s_o0, s_o1]

    def start_idx(g):
        off = base + g * CHUNK
        return pltpu.async_copy(tok_hbm.at[pl.ds(off, CHUNK)], idxb[g & 1],
                                s_i[g & 1])

    def start_gather(g):
        b = g & 1
        return [pltpu.async_copy(
                    tab_hbm.at[idxb[b].at[pl.ds(j * SUB, SUB)]],
                    rowsb[b].at[pl.ds(j * SUB, SUB)],
                    s_g[b])
                for j in range(KSUB)]

    def start_out(g):
        off = base + g * CHUNK
        return pltpu.async_copy(rowsb[g & 1], out_hbm.at[pl.ds(off, CHUNK)],
                                s_o[g & 1])

    ic = [start_idx(0), start_idx(1)]
    ic[0].wait()
    gc = [None, None]
    gc[0] = start_gather(0)
    oc = [None, None]
    for g in range(NUM_CHUNKS):
        b = g & 1
        nb = 1 - b
        if g + 1 < NUM_CHUNKS:
            ic[nb].wait()
            if oc[nb] is not None:
                oc[nb].wait()
            gc[nb] = start_gather(g + 1)
        for c in gc[b]:
            c.wait()
        if g + 2 < NUM_CHUNKS:
            ic[b] = start_idx(g + 2)
        oc[b] = start_out(g)
    oc[(NUM_CHUNKS - 2) & 1].wait()
    oc[(NUM_CHUNKS - 1) & 1].wait()


@jax.jit
def _embedding_lookup(tok_flat, embedding_mat):
    mesh = plsc.VectorSubcoreMesh(core_axis_name="c", subcore_axis_name="s")
    run = pl.kernel(
        _gather_body,
        mesh=mesh,
        out_type=jax.ShapeDtypeStruct((N_TOKENS, EMB_DIM), jnp.float32),
        scratch_types=[
            pltpu.VMEM((CHUNK,), jnp.int32),
            pltpu.VMEM((CHUNK,), jnp.int32),
            pltpu.VMEM((CHUNK, EMB_DIM), jnp.float32),
            pltpu.VMEM((CHUNK, EMB_DIM), jnp.float32),
            pltpu.SemaphoreType.DMA,
            pltpu.SemaphoreType.DMA,
            pltpu.SemaphoreType.DMA,
            pltpu.SemaphoreType.DMA,
            pltpu.SemaphoreType.DMA,
            pltpu.SemaphoreType.DMA,
        ],
        compiler_params=pltpu.CompilerParams(use_tc_tiling_on_sc=False),
    )
    return run(tok_flat, embedding_mat)


def kernel(token_ids, embedding_mat):
    tok_flat = token_ids.reshape(-1).astype(jnp.int32)
    out = _embedding_lookup(tok_flat, embedding_mat)
    return out.reshape(token_ids.shape + (EMB_DIM,))


# speedup vs baseline: 1.2732x; 1.1441x over previous
import functools

import jax
import jax.numpy as jnp
from jax import lax
from jax.experimental import pallas as pl
from jax.experimental.pallas import tpu as pltpu
from jax.experimental.pallas import tpu_sc as plsc

BATCH = 16384
HIST_LEN = 50
EMB_DIM = 32
N_TOKENS = BATCH * HIST_LEN
N_ROWS = 1_000_000

NUM_WORKERS = 32
PER_WORKER = N_TOKENS // NUM_WORKERS
CHUNK = 1600
NUM_CHUNKS = PER_WORKER // CHUNK

QS = 1 << 18
TC_C = 2048
TC_GRID = QS // TC_C
_LAST_BLK = (N_ROWS - 1) // TC_C


def _tc_transpose_body(x0_ref, x1_ref, x2_ref, x3_ref, o_ref):
    for k, xk in enumerate((x0_ref, x1_ref, x2_ref, x3_ref)):
        o_ref[:, k * EMB_DIM:(k + 1) * EMB_DIM] = jnp.transpose(xk[...])


def _gather_body(tok_hbm, tab_hbm, out_hbm,
                 idx0, idx1, rows0, rows1,
                 s_i0, s_i1, s_g0, s_g1, s_o0, s_o1):
    wid = lax.axis_index("s") * 2 + lax.axis_index("c")
    base = wid * PER_WORKER
    idxb, rowsb = [idx0, idx1], [rows0, rows1]
    s_i, s_g, s_o = [s_i0, s_i1], [s_g0, s_g1], [s_o0, s_o1]

    def start_idx(g):
        off = base + g * CHUNK
        return pltpu.async_copy(tok_hbm.at[pl.ds(off, CHUNK)], idxb[g & 1],
                                s_i[g & 1])

    def remap_idx(g):
        ref = idxb[g & 1]

        @pl.loop(0, CHUNK // 16, unroll=4)
        def _(m):
            v = ref[pl.ds(m * 16, 16)]
            ref[pl.ds(m * 16, 16)] = ((v & (QS - 1)) << 2) | (v >> 18)

    def start_gather(g):
        return pltpu.async_copy(tab_hbm.at[idxb[g & 1]], rowsb[g & 1],
                                s_g[g & 1])

    def start_out(g):
        off = base + g * CHUNK
        return pltpu.async_copy(rowsb[g & 1], out_hbm.at[pl.ds(off, CHUNK)],
                                s_o[g & 1])

    ic = [start_idx(0), start_idx(1)]
    ic[0].wait()
    remap_idx(0)
    gc = [None, None]
    gc[0] = start_gather(0)
    oc = [None, None]
    for g in range(NUM_CHUNKS):
        b = g & 1
        nb = 1 - b
        if g + 1 < NUM_CHUNKS:
            ic[nb].wait()
            remap_idx(g + 1)
            if oc[nb] is not None:
                oc[nb].wait()
            gc[nb] = start_gather(g + 1)
        gc[b].wait()
        if g + 2 < NUM_CHUNKS:
            ic[b] = start_idx(g + 2)
        oc[b] = start_out(g)
    oc[(NUM_CHUNKS - 2) & 1].wait()
    oc[(NUM_CHUNKS - 1) & 1].wait()


@jax.jit
def _embedding_lookup(tok_flat, tabT):
    mesh = plsc.VectorSubcoreMesh(core_axis_name="c", subcore_axis_name="s")
    def _quad_map(k):
        return lambda j: (0, jnp.minimum(k * TC_GRID + j, _LAST_BLK))

    transpose = pl.pallas_call(
        _tc_transpose_body,
        grid=(TC_GRID,),
        in_specs=[pl.BlockSpec((EMB_DIM, TC_C), _quad_map(k))
                  for k in range(4)],
        out_specs=pl.BlockSpec((TC_C, 4 * EMB_DIM), lambda j: (j, 0)),
        out_shape=jax.ShapeDtypeStruct((QS, 4 * EMB_DIM), jnp.float32),
    )
    tab_lin = transpose(tabT, tabT, tabT, tabT)
    gather = pl.kernel(
        _gather_body,
        mesh=mesh,
        out_type=jax.ShapeDtypeStruct((N_TOKENS, EMB_DIM), jnp.float32),
        scratch_types=[
            pltpu.VMEM((CHUNK,), jnp.int32),
            pltpu.VMEM((CHUNK,), jnp.int32),
            pltpu.VMEM((CHUNK, EMB_DIM), jnp.float32),
            pltpu.VMEM((CHUNK, EMB_DIM), jnp.float32),
            pltpu.SemaphoreType.DMA,
            pltpu.SemaphoreType.DMA,
            pltpu.SemaphoreType.DMA,
            pltpu.SemaphoreType.DMA,
            pltpu.SemaphoreType.DMA,
            pltpu.SemaphoreType.DMA,
        ],
        compiler_params=pltpu.CompilerParams(use_tc_tiling_on_sc=False),
    )
    return gather(tok_flat, tab_lin.reshape(4 * QS, EMB_DIM))


def kernel(token_ids, embedding_mat):
    tok_flat = token_ids.reshape(-1).astype(jnp.int32)
    out = _embedding_lookup(tok_flat, embedding_mat.T)
    return out.reshape(token_ids.shape + (EMB_DIM,))


# speedup vs baseline: 1.2829x; 1.0077x over previous
import functools

import jax
import jax.numpy as jnp
from jax import lax
from jax.experimental import pallas as pl
from jax.experimental.pallas import tpu as pltpu
from jax.experimental.pallas import tpu_sc as plsc

BATCH = 16384
HIST_LEN = 50
EMB_DIM = 32
N_TOKENS = BATCH * HIST_LEN
N_ROWS = 1_000_000

NUM_WORKERS = 32
PER_WORKER = N_TOKENS // NUM_WORKERS
CHUNK = 1600
NUM_CHUNKS = PER_WORKER // CHUNK

QS = 1 << 18
TC_C = 4096
TC_GRID = QS // TC_C
_LAST_BLK = (N_ROWS - 1) // TC_C


def _tc_transpose_body(x0_ref, x1_ref, x2_ref, x3_ref, o_ref):
    for k, xk in enumerate((x0_ref, x1_ref, x2_ref, x3_ref)):
        o_ref[:, k * EMB_DIM:(k + 1) * EMB_DIM] = jnp.transpose(xk[...])


def _gather_body(tok_hbm, tab_hbm, out_hbm,
                 idx0, idx1, rows0, rows1,
                 s_i0, s_i1, s_g0, s_g1, s_o0, s_o1):
    wid = lax.axis_index("s") * 2 + lax.axis_index("c")
    base = wid * PER_WORKER
    idxb, rowsb = [idx0, idx1], [rows0, rows1]
    s_i, s_g, s_o = [s_i0, s_i1], [s_g0, s_g1], [s_o0, s_o1]

    def start_idx(g):
        off = base + g * CHUNK
        return pltpu.async_copy(tok_hbm.at[pl.ds(off, CHUNK)], idxb[g & 1],
                                s_i[g & 1])

    def remap_idx(g):
        ref = idxb[g & 1]

        @pl.loop(0, CHUNK // 16, unroll=4)
        def _(m):
            v = ref[pl.ds(m * 16, 16)]
            ref[pl.ds(m * 16, 16)] = ((v & (QS - 1)) << 2) | (v >> 18)

    def start_gather(g):
        return pltpu.async_copy(tab_hbm.at[idxb[g & 1]], rowsb[g & 1],
                                s_g[g & 1])

    def start_out(g):
        off = base + g * CHUNK
        return pltpu.async_copy(rowsb[g & 1], out_hbm.at[pl.ds(off, CHUNK)],
                                s_o[g & 1])

    ic = [start_idx(0), start_idx(1)]
    ic[0].wait()
    remap_idx(0)
    gc = [None, None]
    gc[0] = start_gather(0)
    oc = [None, None]
    for g in range(NUM_CHUNKS):
        b = g & 1
        nb = 1 - b
        if g + 1 < NUM_CHUNKS:
            ic[nb].wait()
            remap_idx(g + 1)
            if oc[nb] is not None:
                oc[nb].wait()
            gc[nb] = start_gather(g + 1)
        gc[b].wait()
        if g + 2 < NUM_CHUNKS:
            ic[b] = start_idx(g + 2)
        oc[b] = start_out(g)
    oc[(NUM_CHUNKS - 2) & 1].wait()
    oc[(NUM_CHUNKS - 1) & 1].wait()


@jax.jit
def _embedding_lookup(tok_flat, tabT):
    mesh = plsc.VectorSubcoreMesh(core_axis_name="c", subcore_axis_name="s")
    def _quad_map(k):
        return lambda j: (0, jnp.minimum(k * TC_GRID + j, _LAST_BLK))

    transpose = pl.pallas_call(
        _tc_transpose_body,
        grid=(TC_GRID,),
        in_specs=[pl.BlockSpec((EMB_DIM, TC_C), _quad_map(k))
                  for k in range(4)],
        out_specs=pl.BlockSpec((TC_C, 4 * EMB_DIM), lambda j: (j, 0)),
        out_shape=jax.ShapeDtypeStruct((QS, 4 * EMB_DIM), jnp.float32),
    )
    tab_lin = transpose(tabT, tabT, tabT, tabT)
    gather = pl.kernel(
        _gather_body,
        mesh=mesh,
        out_type=jax.ShapeDtypeStruct((N_TOKENS, EMB_DIM), jnp.float32),
        scratch_types=[
            pltpu.VMEM((CHUNK,), jnp.int32),
            pltpu.VMEM((CHUNK,), jnp.int32),
            pltpu.VMEM((CHUNK, EMB_DIM), jnp.float32),
            pltpu.VMEM((CHUNK, EMB_DIM), jnp.float32),
            pltpu.SemaphoreType.DMA,
            pltpu.SemaphoreType.DMA,
            pltpu.SemaphoreType.DMA,
            pltpu.SemaphoreType.DMA,
            pltpu.SemaphoreType.DMA,
            pltpu.SemaphoreType.DMA,
        ],
        compiler_params=pltpu.CompilerParams(use_tc_tiling_on_sc=False),
    )
    return gather(tok_flat, tab_lin.reshape(4 * QS, EMB_DIM))


def kernel(token_ids, embedding_mat):
    tok_flat = token_ids.reshape(-1).astype(jnp.int32)
    out = _embedding_lookup(tok_flat, embedding_mat.T)
    return out.reshape(token_ids.shape + (EMB_DIM,))


# speedup vs baseline: 2.5558x; 1.9922x over previous
import functools

import jax
import jax.numpy as jnp
from jax import lax
from jax.experimental import pallas as pl
from jax.experimental.pallas import tpu as pltpu
from jax.experimental.pallas import tpu_sc as plsc

BATCH = 16384
HIST_LEN = 50
EMB_DIM = 32
N_TOKENS = BATCH * HIST_LEN
N_ROWS = 1_000_000

NUM_WORKERS = 32
PER_WORKER = N_TOKENS // NUM_WORKERS
CHUNK = 1600
NUM_CHUNKS = PER_WORKER // CHUNK

QS = 1 << 18
TC_C = 4096
TC_GRID = QS // TC_C
_LAST_BLK = (N_ROWS - 1) // TC_C


def _tc_transpose_body(x0_ref, x1_ref, x2_ref, x3_ref, o_ref):
    for k, xk in enumerate((x0_ref, x1_ref, x2_ref, x3_ref)):
        o_ref[:, k * EMB_DIM:(k + 1) * EMB_DIM] = jnp.transpose(xk[...])


def _gather_body(tok_hbm, tab_hbm, out_hbm,
                 idx0, idx1, rows0, rows1,
                 s_i0, s_i1, s_g0, s_g1, s_o0, s_o1):
    wid = lax.axis_index("s") * 2 + lax.axis_index("c")
    base = wid * PER_WORKER
    idxb, rowsb = [idx0, idx1], [rows0, rows1]
    s_i, s_g, s_o = [s_i0, s_i1], [s_g0, s_g1], [s_o0, s_o1]

    def start_idx(g):
        off = base + g * CHUNK
        return pltpu.async_copy(tok_hbm.at[pl.ds(off, CHUNK)], idxb[g & 1],
                                s_i[g & 1])

    def remap_idx(g):
        ref = idxb[g & 1]

        @pl.loop(0, CHUNK // 16, unroll=4)
        def _(m):
            v = ref[pl.ds(m * 16, 16)]
            ref[pl.ds(m * 16, 16)] = ((v & (QS - 1)) << 2) | (v >> 18)

    def start_gather(g):
        return pltpu.async_copy(tab_hbm.at[idxb[g & 1]], rowsb[g & 1],
                                s_g[g & 1])

    def start_out(g):
        off = base + g * CHUNK
        return pltpu.async_copy(rowsb[g & 1], out_hbm.at[pl.ds(off, CHUNK)],
                                s_o[g & 1])

    ic = [start_idx(0), start_idx(1)]
    ic[0].wait()
    remap_idx(0)
    gc = [None, None]
    gc[0] = start_gather(0)
    oc = [None, None]
    for g in range(NUM_CHUNKS):
        b = g & 1
        nb = 1 - b
        if g + 1 < NUM_CHUNKS:
            ic[nb].wait()
            remap_idx(g + 1)
            if oc[nb] is not None:
                oc[nb].wait()
            gc[nb] = start_gather(g + 1)
        gc[b].wait()
        if g + 2 < NUM_CHUNKS:
            ic[b] = start_idx(g + 2)
        oc[b] = start_out(g)
    oc[(NUM_CHUNKS - 2) & 1].wait()
    oc[(NUM_CHUNKS - 1) & 1].wait()


@jax.jit
def _embedding_lookup(tok_flat, tabT):
    mesh = plsc.VectorSubcoreMesh(core_axis_name="c", subcore_axis_name="s")
    def _quad_map(k):
        return lambda j: (0, jnp.minimum(k * TC_GRID + j, _LAST_BLK))

    transpose = pl.pallas_call(
        _tc_transpose_body,
        grid=(TC_GRID,),
        in_specs=[pl.BlockSpec((EMB_DIM, TC_C), _quad_map(k))
                  for k in range(4)],
        out_specs=pl.BlockSpec((TC_C, 4 * EMB_DIM), lambda j: (j, 0)),
        out_shape=jax.ShapeDtypeStruct((QS, 4 * EMB_DIM), jnp.float32),
    )
    tab_lin = transpose(tabT, tabT, tabT, tabT)
    gather = pl.kernel(
        _gather_body,
        mesh=mesh,
        out_type=jax.ShapeDtypeStruct((N_TOKENS, EMB_DIM), jnp.float32),
        scratch_types=[
            pltpu.VMEM((CHUNK,), jnp.int32),
            pltpu.VMEM((CHUNK,), jnp.int32),
            pltpu.VMEM((CHUNK, EMB_DIM), jnp.float32),
            pltpu.VMEM((CHUNK, EMB_DIM), jnp.float32),
            pltpu.SemaphoreType.DMA,
            pltpu.SemaphoreType.DMA,
            pltpu.SemaphoreType.DMA,
            pltpu.SemaphoreType.DMA,
            pltpu.SemaphoreType.DMA,
            pltpu.SemaphoreType.DMA,
        ],
        compiler_params=pltpu.CompilerParams(use_tc_tiling_on_sc=False),
    )
    return gather(tok_flat, tab_lin.reshape(4 * QS, EMB_DIM))


def kernel(token_ids, embedding_mat):
    tok_flat = token_ids.T.reshape(-1).astype(jnp.int32)
    out = _embedding_lookup(tok_flat, embedding_mat.T)
    return out.reshape(HIST_LEN, BATCH, EMB_DIM).transpose(1, 0, 2)
